# baseline (speedup 1.0000x reference)
import functools

import jax
import jax.numpy as jnp
from jax import lax
from jax.experimental import pallas as pl
from jax.experimental.pallas import tpu as pltpu
from jax.experimental.pallas import tpu_sc as plsc

BATCH = 64
CHANNELS = 3
SPATIAL = 512
N = SPATIAL * SPATIAL
P = BATCH * CHANNELS
NUM_CORES = 2
NUM_SUBCORES = 16
CHUNK = N // NUM_SUBCORES
PLANES_PER_CORE = P // NUM_CORES
PAIRS = PLANES_PER_CORE // 2


def _shuffle(x2, perm):
    mesh = plsc.VectorSubcoreMesh(core_axis_name="c", subcore_axis_name="s")

    @functools.partial(
        pl.kernel,
        mesh=mesh,
        out_type=jax.ShapeDtypeStruct((P, N), jnp.float32),
        scratch_types=[
            pltpu.VMEM((CHUNK // 4,), jnp.int32),
            pltpu.VMEM((CHUNK // 4,), jnp.int32),
            pltpu.VMEM((CHUNK // 4,), jnp.int32),
            pltpu.VMEM((CHUNK // 4,), jnp.int32),
            pltpu.VMEM((CHUNK,), jnp.float32),
            pltpu.VMEM((CHUNK,), jnp.float32),
            pltpu.VMEM_SHARED((N,), jnp.float32),
            pltpu.VMEM_SHARED((N,), jnp.float32),
            pltpu.SemaphoreType.DMA,
            pltpu.SemaphoreType.DMA,
            pltpu.SemaphoreType.DMA,
        ],
    )
    def k(x_hbm, perm_hbm, out_hbm, idx0, idx1, idx2, idx3, out0, out1,
          sh0, sh1, sem_l, sem_g, sem_s):
        cid = lax.axis_index("c")
        sid = lax.axis_index("s")
        col0 = sid * CHUNK
        base = cid * PLANES_PER_CORE
        Q = CHUNK // 4
        idx_parts = (idx0, idx1, idx2, idx3)
        for q in range(4):
            pltpu.sync_copy(perm_hbm.at[pl.ds(col0 + q * Q, Q)], idx_parts[q])

        def load(p, sh):
            return pltpu.make_async_copy(
                x_hbm.at[p, pl.ds(col0, CHUNK)],
                sh.at[pl.ds(col0, CHUNK)],
                sem_l,
            )

        def store(p, out_v):
            return pltpu.make_async_copy(
                out_v, out_hbm.at[p, pl.ds(col0, CHUNK)], sem_s
            )

        load(base, sh0).start()

        def half(j, p, sh_cur, sh_nxt, out_v, issue_next):
            load(p, sh_cur).wait()
            plsc.subcore_barrier()

            @pl.when(issue_next)
            def _():
                load(p + 1, sh_nxt).start()

            @pl.when(j >= 1)
            def _():
                store(p, out_v).wait()

            copies = [
                pltpu.make_async_copy(
                    sh_cur.at[idx_parts[q]], out_v.at[pl.ds(q * Q, Q)], sem_g
                )
                for q in range(4)
            ]
            for c in copies:
                c.start()
            for c in copies:
                c.wait()
            store(p, out_v).start()

        def body(j, _):
            a = base + 2 * j
            half(j, a, sh0, sh1, out0, jnp.bool_(True))
            half(j, a + 1, sh1, sh0, out1, j + 1 < PAIRS)
            return ()

        lax.fori_loop(0, PAIRS, body, ())
        store(base, out0).wait()
        store(base, out1).wait()

    return k(x2, perm)


def kernel(x, permutation):
    x2 = x.reshape(P, N)
    perm = permutation.astype(jnp.int32)
    out = _shuffle(x2, perm)
    return out.reshape(x.shape)


# speedup vs baseline: 1.0232x; 1.0232x over previous
import functools

import jax
import jax.numpy as jnp
from jax import lax
from jax.experimental import pallas as pl
from jax.experimental.pallas import tpu as pltpu
from jax.experimental.pallas import tpu_sc as plsc

BATCH = 64
CHANNELS = 3
SPATIAL = 512
N = SPATIAL * SPATIAL
P = BATCH * CHANNELS
NUM_CORES = 2
NUM_SUBCORES = 16
CHUNK = N // NUM_SUBCORES
PP = P // NUM_CORES


def _shuffle(x2, perm):
    mesh = plsc.VectorSubcoreMesh(core_axis_name="c", subcore_axis_name="s")

    @functools.partial(
        pl.kernel,
        mesh=mesh,
        out_type=jax.ShapeDtypeStruct((P, N), jnp.float32),
        scratch_types=[
            pltpu.VMEM((CHUNK,), jnp.int32),
            pltpu.VMEM((CHUNK,), jnp.float32),
            pltpu.VMEM((CHUNK,), jnp.float32),
            pltpu.VMEM_SHARED((N,), jnp.float32),
            pltpu.VMEM_SHARED((N,), jnp.float32),
            pltpu.VMEM_SHARED((N,), jnp.float32),
            pltpu.SemaphoreType.DMA,
            pltpu.SemaphoreType.DMA,
            pltpu.SemaphoreType.DMA,
        ],
    )
    def k(x_hbm, perm_hbm, out_hbm, idx_v, out0, out1, sh0, sh1, sh2,
          sem_l, sem_g, sem_s):
        cid = lax.axis_index("c")
        sid = lax.axis_index("s")
        col0 = sid * CHUNK
        base = cid * PP
        pltpu.sync_copy(perm_hbm.at[pl.ds(col0, CHUNK)], idx_v)

        outs = (out0, out1)

        def load(p, sh):
            return pltpu.make_async_copy(
                x_hbm.at[p, pl.ds(col0, CHUNK)],
                sh.at[pl.ds(col0, CHUNK)],
                sem_l,
            )

        def gather(sh, out_v):
            return pltpu.make_async_copy(sh.at[idx_v], out_v, sem_g)

        def store(p, out_v):
            return pltpu.make_async_copy(
                out_v, out_hbm.at[p, pl.ds(col0, CHUNK)], sem_s
            )

        load(base, sh0).start()
        load(base + 1, sh1).start()
        load(base, sh0).wait()
        plsc.subcore_barrier()
        gather(sh0, out0).start()

        def step(i, sh_cur, sh_nxt, sh_nxt2, out_cur, out_nxt):
            del sh_cur
            @pl.when(i + 1 < base + PP)
            def _():
                load(i + 1, sh_nxt).wait()

            @pl.when(i >= base + 1)
            def _():
                store(i - 1, out_nxt).wait()

            plsc.subcore_barrier()

            @pl.when(i + 1 < base + PP)
            def _():
                gather(sh_nxt, out_nxt).start()

            @pl.when(i + 2 < base + PP)
            def _():
                load(i + 2, sh_nxt2).start()

            pltpu.make_async_copy(sh_nxt.at[idx_v], out_cur, sem_g).wait()
            store(i, out_cur).start()

        def body(j, _):
            i = base + 6 * j
            sh = (sh0, sh1, sh2)
            for u in range(6):
                step(
                    i + u,
                    sh[u % 3],
                    sh[(u + 1) % 3],
                    sh[(u + 2) % 3],
                    outs[u % 2],
                    outs[(u + 1) % 2],
                )
            return ()

        lax.fori_loop(0, PP // 6, body, ())
        store(base, outs[(PP - 1) % 2]).wait()

    return k(x2, perm)


def kernel(x, permutation):
    x2 = x.reshape(P, N)
    perm = permutation.astype(jnp.int32)
    out = _shuffle(x2, perm)
    return out.reshape(x.shape)
